# Initial kernel scaffold; baseline (speedup 1.0000x reference)
import jax
import jax.numpy as jnp
from jax.experimental import pallas as pl


def kernel(output, targets):
    raise NotImplementedError("write your pallas kernel here")


# speedup vs baseline: 2.0552x; 2.0552x over previous
import math

import jax
import jax.numpy as jnp
from jax.experimental import pallas as pl
from jax.experimental.pallas import tpu as pltpu

_VOCAB = 100000
_BATCH = 1024
_SMOOTHING = 0.1
_CONF = 1.0 - _SMOOTHING
_SMOOTH = _SMOOTHING / (_VOCAB - 2)
_WRAP = _VOCAB - 100

_BLOCK_V = 4096
_GRID = -(-_VOCAB // _BLOCK_V)

_S_LOG_S = float(_SMOOTH * math.log(_SMOOTH))
_CONST = float(_BATCH * (_CONF * math.log(_CONF)
                         + (_VOCAB - 2) * _SMOOTH * math.log(_SMOOTH)))


def _loss_kernel(out_ref, tgt_ref, loss_ref):
    j = pl.program_id(0)
    cols = j * _BLOCK_V + jax.lax.broadcasted_iota(
        jnp.int32, (_BATCH, _BLOCK_V), 1)
    t = tgt_ref[...]
    is_t = cols == t
    is_w = cols == _WRAP
    valid = cols < _VOCAB
    p = jnp.where(is_t, _CONF, jnp.where(is_w, 0.0, _SMOOTH))
    p = jnp.where(valid, p, 0.0)
    x = jnp.where(valid, out_ref[...], 0.0)
    dot = jnp.sum(p * x)
    n_w = jnp.sum(jnp.where(is_t & is_w, 1.0, 0.0))
    contrib = n_w * _S_LOG_S - dot

    @pl.when(j == 0)
    def _init():
        loss_ref[0, 0] = _CONST

    loss_ref[0, 0] += contrib


def kernel(output, targets):
    tgt = targets.reshape(_BATCH, 1)
    loss = pl.pallas_call(
        _loss_kernel,
        grid=(_GRID,),
        in_specs=[
            pl.BlockSpec((_BATCH, _BLOCK_V), lambda j: (0, j)),
            pl.BlockSpec((_BATCH, 1), lambda j: (0, 0)),
        ],
        out_specs=pl.BlockSpec((1, 1), lambda j: (0, 0),
                               memory_space=pltpu.SMEM),
        out_shape=jax.ShapeDtypeStruct((1, 1), jnp.float32),
        compiler_params=pltpu.CompilerParams(
            dimension_semantics=("arbitrary",)),
    )(output, tgt)
    return loss[0, 0]


# speedup vs baseline: 2.4002x; 1.1679x over previous
import math

import jax
import jax.numpy as jnp
from jax.experimental import pallas as pl
from jax.experimental.pallas import tpu as pltpu

_VOCAB = 100000
_BATCH = 1024
_SMOOTHING = 0.1
_CONF = 1.0 - _SMOOTHING
_SMOOTH = _SMOOTHING / (_VOCAB - 2)
_WRAP = _VOCAB - 100

_BLOCK_V = 4096
_GRID = -(-_VOCAB // _BLOCK_V)
_JW = _WRAP // _BLOCK_V
_WOFF = _WRAP - _JW * _BLOCK_V
_SCALE = _CONF / _SMOOTH

_S_LOG_S = float(_SMOOTH * math.log(_SMOOTH))
_CONST = float(_BATCH * (_CONF * math.log(_CONF)
                         + (_VOCAB - 2) * _SMOOTH * math.log(_SMOOTH)))


def _loss_kernel(out_ref, tgt_ref, loss_ref, acc_ref):
    j = pl.program_id(0)

    @pl.when(j == 0)
    def _init():
        acc_ref[...] = jnp.zeros_like(acc_ref)
        loss_ref[0, 0] = _CONST

    x = out_ref[...]
    t = tgt_ref[...]
    cols = j * _BLOCK_V + jax.lax.broadcasted_iota(
        jnp.int32, (_BATCH, _BLOCK_V), 1)
    is_t = cols == t
    ones = jnp.ones((1, _BATCH), dtype=jnp.float32)

    @pl.when(j < _GRID - 1)
    def _full():
        z = jnp.where(is_t, x * _SCALE, x)
        acc_ref[...] += jax.lax.dot_general(
            ones, z, (((1,), (0,)), ((), ())),
            preferred_element_type=jnp.float32)

    @pl.when(j == _GRID - 1)
    def _tail():
        valid = cols < _VOCAB
        z = jnp.where(is_t, x * _SCALE, jnp.where(valid, x, 0.0))
        acc_ref[...] += jax.lax.dot_general(
            ones, z, (((1,), (0,)), ((), ())),
            preferred_element_type=jnp.float32)

    @pl.when(j == _JW)
    def _wrap_fix():
        colw = x[:, _WOFF:_WOFF + 1]
        t_is_w = t == _WRAP
        w_all = jnp.sum(colw)
        w_keep = jnp.sum(jnp.where(t_is_w, colw, 0.0))
        n_w = jnp.sum(jnp.where(t_is_w, 1.0, 0.0))
        loss_ref[0, 0] += n_w * _S_LOG_S + _SMOOTH * (w_all - w_keep)

    @pl.when(j == _GRID - 1)
    def _finish():
        loss_ref[0, 0] += -_SMOOTH * jnp.sum(acc_ref[...])


def kernel(output, targets):
    tgt = targets.reshape(_BATCH, 1)
    loss = pl.pallas_call(
        _loss_kernel,
        grid=(_GRID,),
        in_specs=[
            pl.BlockSpec((_BATCH, _BLOCK_V), lambda j: (0, j)),
            pl.BlockSpec((_BATCH, 1), lambda j: (0, 0)),
        ],
        out_specs=pl.BlockSpec((1, 1), lambda j: (0, 0),
                               memory_space=pltpu.SMEM),
        out_shape=jax.ShapeDtypeStruct((1, 1), jnp.float32),
        scratch_shapes=[pltpu.VMEM((1, _BLOCK_V), jnp.float32)],
        compiler_params=pltpu.CompilerParams(
            dimension_semantics=("arbitrary",)),
    )(output, tgt)
    return loss[0, 0]
